# Initial kernel scaffold; baseline (speedup 1.0000x reference)
import jax
import jax.numpy as jnp
from jax.experimental import pallas as pl


def kernel(x, edge_index, batch, W1, b1, W2, b2, Mw1, Mb1, Mw2, Mb2):
    raise NotImplementedError("write your pallas kernel here")


# speedup vs baseline: 1.8121x; 1.8121x over previous
import functools

import jax
import jax.numpy as jnp
from jax.experimental import pallas as pl
from jax.experimental.pallas import tpu as pltpu

N = 10000
E = 320000
D_IN = 128
H = 128
OUT = 3
NUM_HE = 10000
NG = 64


def _matmul_body(x_ref, w_ref, o_ref):
    o_ref[...] = jnp.dot(x_ref[...], w_ref[...],
                         preferred_element_type=jnp.float32)


def _tc_matmul(x, w):
    m, k = x.shape
    _, n = w.shape
    return pl.pallas_call(
        _matmul_body,
        out_shape=jax.ShapeDtypeStruct((m, n), jnp.float32),
    )(x, w)


def kernel(x, edge_index, batch, W1, b1, W2, b2, Mw1, Mb1, Mw2, Mb2):
    src = edge_index[0]
    he = edge_index[1]
    ones = jnp.ones((E,), dtype=jnp.float32)
    Ddeg = jax.ops.segment_sum(ones, src, num_segments=N)
    Dinv = jnp.where(Ddeg > 0, 1.0 / Ddeg, 0.0)
    Bdeg = jax.ops.segment_sum(ones, he, num_segments=NUM_HE)
    Binv = jnp.where(Bdeg > 0, 1.0 / Bdeg, 0.0)

    def hg(h_in, W, b):
        xw = _tc_matmul(h_in, W)
        he_feat = jax.ops.segment_sum(xw[src], he, num_segments=NUM_HE)
        he_feat = he_feat * Binv[:, None]
        out = jax.ops.segment_sum(he_feat[he], src, num_segments=N)
        return out * Dinv[:, None] + b

    h1 = jax.nn.relu(hg(x, W1, b1))
    h2 = jax.nn.relu(hg(h1, W2, b2))
    counts = jax.ops.segment_sum(jnp.ones((N,), jnp.float32), batch,
                                 num_segments=NG)
    pooled = jax.ops.segment_sum(h2, batch, num_segments=NG)
    pooled = pooled / jnp.maximum(counts, 1.0)[:, None]
    out = jax.nn.relu(_tc_matmul(pooled, Mw1) + Mb1)
    out = _tc_matmul(out, Mw2) + Mb2
    return out


# speedup vs baseline: 13.1030x; 7.2310x over previous
import functools

import jax
import jax.numpy as jnp
from jax import lax
from jax.experimental import pallas as pl
from jax.experimental.pallas import tpu as pltpu
from jax.experimental.pallas import tpu_sc as plsc

N = 10000
E = 320000
D_IN = 128
H = 128
OUT = 3
NUM_HE = 10000
NG = 64

_NC = 2
_NS = 16
_NW = _NC * _NS
_EPW = E // _NW
_CH = 125
_NCHK = _EPW // _CH
_NP = 10240
_RPT = _NP // _NS

_mesh = plsc.VectorSubcoreMesh(core_axis_name="c", subcore_axis_name="s")


def _spmm_body(table, idxg, idxs, zeros, out, acc, idxg_v, idxs_v, rows, sem):
    c = lax.axis_index("c")
    s = lax.axis_index("s")
    wid = c * _NS + s
    pltpu.sync_copy(zeros, acc.at[pl.ds(s * _RPT, _RPT)])
    pltpu.sync_copy(idxg.at[wid], idxg_v)
    pltpu.sync_copy(idxs.at[wid], idxs_v)
    plsc.subcore_barrier()

    def body(j, carry):
        pltpu.async_copy(table.at[idxg_v.at[j]], rows, sem).wait()
        pltpu.sync_copy(rows, acc.at[idxs_v.at[j]], add=True)
        return carry

    lax.fori_loop(0, _NCHK, body, 0)
    plsc.subcore_barrier()
    base = c * _NP + s * _RPT
    pltpu.sync_copy(acc.at[pl.ds(s * _RPT, _RPT)], out.at[pl.ds(base, _RPT)])


def _sc_spmm(table, idxg, idxs, zeros):
    return pl.kernel(
        _spmm_body,
        out_type=jax.ShapeDtypeStruct((_NC * _NP, H), jnp.float32),
        mesh=_mesh,
        scratch_types=[
            pltpu.VMEM_SHARED((_NP, H), jnp.float32),
            pltpu.VMEM((_NCHK, _CH), jnp.int32),
            pltpu.VMEM((_NCHK, _CH), jnp.int32),
            pltpu.VMEM((_CH, H), jnp.float32),
            pltpu.SemaphoreType.DMA,
        ],
    )(table, idxg, idxs, zeros)


def _deg_body(idxd, idxb, ones_h, zeros16, outd, outb,
              accd, accb, idxd_v, idxb_v, ones_v):
    c = lax.axis_index("c")
    s = lax.axis_index("s")
    wid = c * _NS + s
    pltpu.sync_copy(zeros16, accd.at[pl.ds(s * _RPT, _RPT)])
    pltpu.sync_copy(zeros16, accb.at[pl.ds(s * _RPT, _RPT)])
    pltpu.sync_copy(ones_h, ones_v)
    pltpu.sync_copy(idxd.at[wid], idxd_v)
    pltpu.sync_copy(idxb.at[wid], idxb_v)
    plsc.subcore_barrier()

    def body(j, carry):
        pltpu.sync_copy(ones_v, accd.at[idxd_v.at[j]], add=True)
        pltpu.sync_copy(ones_v, accb.at[idxb_v.at[j]], add=True)
        return carry

    lax.fori_loop(0, _NCHK, body, 0)
    plsc.subcore_barrier()
    base = c * _NP + s * _RPT
    pltpu.sync_copy(accd.at[pl.ds(s * _RPT, _RPT)], outd.at[pl.ds(base, _RPT)])
    pltpu.sync_copy(accb.at[pl.ds(s * _RPT, _RPT)], outb.at[pl.ds(base, _RPT)])


def _sc_degrees(idxd, idxb, ones16, zeros16):
    return pl.kernel(
        _deg_body,
        out_type=(jax.ShapeDtypeStruct((_NC * _NP, 16), jnp.float32),
                  jax.ShapeDtypeStruct((_NC * _NP, 16), jnp.float32)),
        mesh=_mesh,
        scratch_types=[
            pltpu.VMEM_SHARED((_NP, 16), jnp.float32),
            pltpu.VMEM_SHARED((_NP, 16), jnp.float32),
            pltpu.VMEM((_NCHK, _CH), jnp.int32),
            pltpu.VMEM((_NCHK, _CH), jnp.int32),
            pltpu.VMEM((_CH, 16), jnp.float32),
        ],
    )(idxd, idxb, ones16, zeros16)


def _mm_body(x_ref, w_ref, o_ref):
    o_ref[...] = jnp.dot(x_ref[...], w_ref[...],
                         preferred_element_type=jnp.float32)


def _tc_matmul(x, w):
    m, _ = x.shape
    _, n = w.shape
    return pl.pallas_call(
        _mm_body,
        out_shape=jax.ShapeDtypeStruct((m, n), jnp.float32),
    )(x, w)


def _combine_scale_body(p_ref, deg_ref, o_ref):
    p = p_ref[0:N, :] + p_ref[_NP:_NP + N, :]
    d = deg_ref[0:N, 0:1] + deg_ref[_NP:_NP + N, 0:1]
    inv = jnp.where(d > 0, 1.0 / d, 0.0)
    o_ref[...] = p * inv


def _tc_combine_scale(p, deg):
    return pl.pallas_call(
        _combine_scale_body,
        out_shape=jax.ShapeDtypeStruct((N, H), jnp.float32),
    )(p, deg)


def _finish_mm_body(q_ref, deg_ref, b_ref, w_ref, o_ref):
    q = q_ref[0:N, :] + q_ref[_NP:_NP + N, :]
    d = deg_ref[0:N, 0:1] + deg_ref[_NP:_NP + N, 0:1]
    inv = jnp.where(d > 0, 1.0 / d, 0.0)
    h1 = jax.nn.relu(q * inv + b_ref[...])
    o_ref[...] = jnp.dot(h1, w_ref[...], preferred_element_type=jnp.float32)


def _tc_finish_mm(q, deg, b, w):
    return pl.pallas_call(
        _finish_mm_body,
        out_shape=jax.ShapeDtypeStruct((N, H), jnp.float32),
    )(q, deg, b, w)


def _tail_body(q_ref, deg_ref, b_ref, batch_ref, mw1_ref, mb1_ref,
               mw2_ref, mb2_ref, o_ref):
    q = q_ref[0:N, :] + q_ref[_NP:_NP + N, :]
    d = deg_ref[0:N, 0:1] + deg_ref[_NP:_NP + N, 0:1]
    inv = jnp.where(d > 0, 1.0 / d, 0.0)
    h2 = jax.nn.relu(q * inv + b_ref[...])
    groups = lax.broadcasted_iota(jnp.int32, (NG, N), 0)
    oh = (batch_ref[...] == groups).astype(jnp.float32)
    pooled = jnp.dot(oh, h2, preferred_element_type=jnp.float32)
    counts = jnp.sum(oh, axis=1, keepdims=True)
    pooled = pooled / jnp.maximum(counts, 1.0)
    z = jax.nn.relu(jnp.dot(pooled, mw1_ref[...],
                            preferred_element_type=jnp.float32) + mb1_ref[...])
    o_ref[...] = jnp.dot(z, mw2_ref[...],
                         preferred_element_type=jnp.float32) + mb2_ref[...]


def _tc_tail(q, deg, b, batch2d, mw1, mb1, mw2, mb2):
    return pl.pallas_call(
        _tail_body,
        out_shape=jax.ShapeDtypeStruct((NG, OUT), jnp.float32),
    )(q, deg, b, batch2d, mw1, mb1, mw2, mb2)


def kernel(x, edge_index, batch, W1, b1, W2, b2, Mw1, Mb1, Mw2, Mb2):
    src = edge_index[0].reshape(_NW, _NCHK, _CH)
    he = edge_index[1].reshape(_NW, _NCHK, _CH)
    zeros = jnp.zeros((_RPT, H), jnp.float32)
    zeros16 = jnp.zeros((_RPT, 16), jnp.float32)
    ones16 = jnp.ones((_CH, 16), jnp.float32)
    batch2d = batch.reshape(1, N)
    b1r = b1.reshape(1, H)
    b2r = b2.reshape(1, H)
    mb1r = Mb1.reshape(1, H)
    mb2r = Mb2.reshape(1, OUT)

    degd_p, degb_p = _sc_degrees(src, he, ones16, zeros16)

    xw1 = _tc_matmul(x, W1)
    p1 = _sc_spmm(xw1, src, he, zeros)
    he1 = _tc_combine_scale(p1, degb_p)
    q1 = _sc_spmm(he1, he, src, zeros)
    xw2 = _tc_finish_mm(q1, degd_p, b1r, W2)
    p2 = _sc_spmm(xw2, src, he, zeros)
    he2 = _tc_combine_scale(p2, degb_p)
    q2 = _sc_spmm(he2, he, src, zeros)
    return _tc_tail(q2, degd_p, b2r, batch2d, Mw1, mb1r, Mw2, mb2r)


# speedup vs baseline: 17.7597x; 1.3554x over previous
import functools

import jax
import jax.numpy as jnp
from jax import lax
from jax.experimental import pallas as pl
from jax.experimental.pallas import tpu as pltpu
from jax.experimental.pallas import tpu_sc as plsc

N = 10000
E = 320000
D_IN = 128
H = 128
OUT = 3
NUM_HE = 10000
NG = 64

_NC = 2
_NS = 16
_NW = _NC * _NS
_EPW = E // _NW
_CH = 80
_NCHK = _EPW // _CH
_NP = 10240
_RPT = _NP // _NS

_mesh = plsc.VectorSubcoreMesh(core_axis_name="c", subcore_axis_name="s")


def _spmm_body(table, idxg, idxs, zeros, out, acc, idxg_v, idxs_v,
               rows0, rows1, sem0, sem1):
    c = lax.axis_index("c")
    s = lax.axis_index("s")
    wid = c * _NS + s
    pltpu.sync_copy(zeros, acc.at[pl.ds(s * _RPT, _RPT)])
    pltpu.sync_copy(idxg.at[pl.ds(wid * _EPW, _EPW)], idxg_v)
    pltpu.sync_copy(idxs.at[wid], idxs_v)
    plsc.subcore_barrier()

    npair = (_NCHK - 1) // 2
    def _g(j):
        return table.at[idxg_v.at[pl.ds(j * _CH, _CH)]]

    pltpu.async_copy(_g(0), rows0, sem0)
    pltpu.async_copy(_g(1), rows1, sem1)
    pltpu.make_async_copy(_g(0), rows0, sem0).wait()
    pltpu.sync_copy(rows0, acc.at[idxs_v.at[0]], add=True)
    pltpu.async_copy(_g(2), rows0, sem0)

    def body(k, carry):
        j0 = 2 * k + 1
        j1 = 2 * k + 2
        pltpu.make_async_copy(_g(j0), rows1, sem1).wait()
        pltpu.sync_copy(rows1, acc.at[idxs_v.at[j0]], add=True)

        @pl.when(k < npair - 1)
        def _():
            pltpu.async_copy(_g(j0 + 2), rows1, sem1)

        pltpu.make_async_copy(_g(j1), rows0, sem0).wait()
        pltpu.sync_copy(rows0, acc.at[idxs_v.at[j1]], add=True)

        @pl.when(k < npair - 1)
        def _():
            pltpu.async_copy(_g(j1 + 2), rows0, sem0)

        return carry

    lax.fori_loop(0, npair, body, 0)
    plsc.subcore_barrier()
    base = c * _NP + s * _RPT
    pltpu.sync_copy(acc.at[pl.ds(s * _RPT, _RPT)], out.at[pl.ds(base, _RPT)])


def _sc_spmm(table, idxg, idxs, zeros):
    return pl.kernel(
        _spmm_body,
        out_type=jax.ShapeDtypeStruct((_NC * _NP, H), jnp.float32),
        mesh=_mesh,
        scratch_types=[
            pltpu.VMEM_SHARED((_NP, H), jnp.float32),
            pltpu.VMEM((_EPW,), jnp.int32),
            pltpu.VMEM((_NCHK, _CH), jnp.int32),
            pltpu.VMEM((_CH, H), jnp.float32),
            pltpu.VMEM((_CH, H), jnp.float32),
            pltpu.SemaphoreType.DMA,
            pltpu.SemaphoreType.DMA,
        ],
    )(table, idxg, idxs, zeros)


def _deg_body(idxd, idxb, ones_h, zeros16, outd, outb,
              accd, accb, idxd_v, idxb_v, ones_v):
    c = lax.axis_index("c")
    s = lax.axis_index("s")
    wid = c * _NS + s
    pltpu.sync_copy(zeros16, accd.at[pl.ds(s * _RPT, _RPT)])
    pltpu.sync_copy(zeros16, accb.at[pl.ds(s * _RPT, _RPT)])
    pltpu.sync_copy(ones_h, ones_v)
    pltpu.sync_copy(idxd.at[wid], idxd_v)
    pltpu.sync_copy(idxb.at[wid], idxb_v)
    plsc.subcore_barrier()

    def body(j, carry):
        pltpu.sync_copy(ones_v, accd.at[idxd_v.at[j]], add=True)
        pltpu.sync_copy(ones_v, accb.at[idxb_v.at[j]], add=True)
        return carry

    lax.fori_loop(0, _NCHK, body, 0)
    plsc.subcore_barrier()
    base = c * _NP + s * _RPT
    pltpu.sync_copy(accd.at[pl.ds(s * _RPT, _RPT)], outd.at[pl.ds(base, _RPT)])
    pltpu.sync_copy(accb.at[pl.ds(s * _RPT, _RPT)], outb.at[pl.ds(base, _RPT)])


def _sc_degrees(idxd, idxb, ones16, zeros16):
    return pl.kernel(
        _deg_body,
        out_type=(jax.ShapeDtypeStruct((_NC * _NP, 16), jnp.float32),
                  jax.ShapeDtypeStruct((_NC * _NP, 16), jnp.float32)),
        mesh=_mesh,
        scratch_types=[
            pltpu.VMEM_SHARED((_NP, 16), jnp.float32),
            pltpu.VMEM_SHARED((_NP, 16), jnp.float32),
            pltpu.VMEM((_NCHK, _CH), jnp.int32),
            pltpu.VMEM((_NCHK, _CH), jnp.int32),
            pltpu.VMEM((_CH, 16), jnp.float32),
        ],
    )(idxd, idxb, ones16, zeros16)


def _mm_body(x_ref, w_ref, o_ref):
    o_ref[...] = jnp.dot(x_ref[...], w_ref[...],
                         preferred_element_type=jnp.float32)


def _tc_matmul(x, w):
    m, _ = x.shape
    _, n = w.shape
    return pl.pallas_call(
        _mm_body,
        out_shape=jax.ShapeDtypeStruct((m, n), jnp.float32),
    )(x, w)


def _combine_scale_body(p_ref, deg_ref, o_ref):
    p = p_ref[0:N, :] + p_ref[_NP:_NP + N, :]
    d = deg_ref[0:N, 0:1] + deg_ref[_NP:_NP + N, 0:1]
    inv = jnp.where(d > 0, 1.0 / d, 0.0)
    o_ref[...] = p * inv


def _tc_combine_scale(p, deg):
    return pl.pallas_call(
        _combine_scale_body,
        out_shape=jax.ShapeDtypeStruct((N, H), jnp.float32),
    )(p, deg)


def _finish_mm_body(q_ref, deg_ref, b_ref, w_ref, o_ref):
    q = q_ref[0:N, :] + q_ref[_NP:_NP + N, :]
    d = deg_ref[0:N, 0:1] + deg_ref[_NP:_NP + N, 0:1]
    inv = jnp.where(d > 0, 1.0 / d, 0.0)
    h1 = jax.nn.relu(q * inv + b_ref[...])
    o_ref[...] = jnp.dot(h1, w_ref[...], preferred_element_type=jnp.float32)


def _tc_finish_mm(q, deg, b, w):
    return pl.pallas_call(
        _finish_mm_body,
        out_shape=jax.ShapeDtypeStruct((N, H), jnp.float32),
    )(q, deg, b, w)


def _tail_body(q_ref, deg_ref, b_ref, batch_ref, mw1_ref, mb1_ref,
               mw2_ref, mb2_ref, o_ref):
    q = q_ref[0:N, :] + q_ref[_NP:_NP + N, :]
    d = deg_ref[0:N, 0:1] + deg_ref[_NP:_NP + N, 0:1]
    inv = jnp.where(d > 0, 1.0 / d, 0.0)
    h2 = jax.nn.relu(q * inv + b_ref[...])
    groups = lax.broadcasted_iota(jnp.int32, (NG, N), 0)
    oh = (batch_ref[...] == groups).astype(jnp.float32)
    pooled = jnp.dot(oh, h2, preferred_element_type=jnp.float32)
    counts = jnp.sum(oh, axis=1, keepdims=True)
    pooled = pooled / jnp.maximum(counts, 1.0)
    z = jax.nn.relu(jnp.dot(pooled, mw1_ref[...],
                            preferred_element_type=jnp.float32) + mb1_ref[...])
    o_ref[...] = jnp.dot(z, mw2_ref[...],
                         preferred_element_type=jnp.float32) + mb2_ref[...]


def _tc_tail(q, deg, b, batch2d, mw1, mb1, mw2, mb2):
    return pl.pallas_call(
        _tail_body,
        out_shape=jax.ShapeDtypeStruct((NG, OUT), jnp.float32),
    )(q, deg, b, batch2d, mw1, mb1, mw2, mb2)


def kernel(x, edge_index, batch, W1, b1, W2, b2, Mw1, Mb1, Mw2, Mb2):
    src_flat = edge_index[0]
    he_flat = edge_index[1]
    src = edge_index[0].reshape(_NW, _NCHK, _CH)
    he = edge_index[1].reshape(_NW, _NCHK, _CH)
    zeros = jnp.zeros((_RPT, H), jnp.float32)
    zeros16 = jnp.zeros((_RPT, 16), jnp.float32)
    ones16 = jnp.ones((_CH, 16), jnp.float32)
    batch2d = batch.reshape(1, N)
    b1r = b1.reshape(1, H)
    b2r = b2.reshape(1, H)
    mb1r = Mb1.reshape(1, H)
    mb2r = Mb2.reshape(1, OUT)

    degd_p, degb_p = _sc_degrees(src, he, ones16, zeros16)

    xw1 = _tc_matmul(x, W1)
    p1 = _sc_spmm(xw1, src_flat, he, zeros)
    he1 = _tc_combine_scale(p1, degb_p)
    q1 = _sc_spmm(he1, he_flat, src, zeros)
    xw2 = _tc_finish_mm(q1, degd_p, b1r, W2)
    p2 = _sc_spmm(xw2, src_flat, he, zeros)
    he2 = _tc_combine_scale(p2, degb_p)
    q2 = _sc_spmm(he2, he_flat, src, zeros)
    return _tc_tail(q2, degd_p, b2r, batch2d, Mw1, mb1r, Mw2, mb2r)
